# baseline (speedup 1.0000x reference)
import jax
import jax.numpy as jnp
from jax import lax
from jax.experimental import pallas as pl
from jax.experimental.pallas import tpu as pltpu
from jax.experimental.pallas import tpu_sc as plsc

VOCAB = 100000
EMBED_DIM = 64
BATCH = 4096
HIST = 50

NC = 2
NS = 16
NW = NC * NS

CH = BATCH // NW
NSLOT = 3
TSLOT = 2
LOOKAHEAD = 2
LANES = 16


def _transpose_chunk(gbuf_s, tbuf, t, rows):
    def per_f8(f8, carry):
        for fr in range(8):
            col = jnp.full((LANES,), f8 * 8 + fr, jnp.int32)
            for k in range(8):
                v = plsc.load_gather(gbuf_s, [rows[k], col])
                tbuf[t, f8, fr, pl.ds(k * LANES, LANES)] = v
        return carry
    lax.fori_loop(0, 8, per_f8, 0)


def _glove_body(words_h, ctx_h, wemb_h, wbias_h, cemb_h, cbias_h,
                out_we, out_wb, out_ce, out_cb,
                idx_w, idx_c, wbuf, cbuf, wtile, ctile, wbias_v, cbias_v,
                gsem_w, gsem_c, wsem_w, wsem_c, bsem):
    wid = lax.axis_index("s") * NC + lax.axis_index("c")
    b0 = wid * CH

    pltpu.sync_copy(words_h.at[:, pl.ds(b0, CH)], idx_w)
    pltpu.sync_copy(ctx_h.at[:, pl.ds(b0, CH)], idx_c)

    rows = [lax.iota(jnp.int32, 16) + 16 * k for k in range(8)]

    for h in range(LOOKAHEAD):
        pltpu.async_copy(wemb_h.at[idx_w.at[h]], wbuf.at[h], gsem_w.at[h])
        pltpu.async_copy(cemb_h.at[idx_c.at[h]], cbuf.at[h], gsem_c.at[h])

    def step(h, carry):
        s = lax.rem(h, NSLOT)
        t = lax.rem(h, TSLOT)
        pltpu.make_async_copy(wemb_h.at[idx_w.at[h]], wbuf.at[s],
                              gsem_w.at[s]).wait()
        pltpu.make_async_copy(cemb_h.at[idx_c.at[h]], cbuf.at[s],
                              gsem_c.at[s]).wait()
        hn = h + LOOKAHEAD
        sn = lax.rem(hn, NSLOT)

        @pl.when(hn < HIST)
        def _():
            pltpu.async_copy(wemb_h.at[idx_w.at[hn]], wbuf.at[sn],
                             gsem_w.at[sn])
            pltpu.async_copy(cemb_h.at[idx_c.at[hn]], cbuf.at[sn],
                             gsem_c.at[sn])
        pltpu.async_copy(wbias_h.at[idx_w.at[h]], wbias_v.at[h], bsem)
        pltpu.async_copy(cbias_h.at[idx_c.at[h]], cbias_v.at[h], bsem)

        @pl.when(h >= TSLOT)
        def _():
            for f8 in range(8):
                pltpu.make_async_copy(wtile.at[t, f8], out_we.at[0, f8, 0],
                                      wsem_w.at[t]).wait()
                pltpu.make_async_copy(ctile.at[t, f8], out_ce.at[0, f8, 0],
                                      wsem_c.at[t]).wait()

        _transpose_chunk(wbuf.at[s], wtile, t, rows)
        _transpose_chunk(cbuf.at[s], ctile, t, rows)
        for f8 in range(8):
            pltpu.async_copy(wtile.at[t, f8], out_we.at[h, f8, wid],
                             wsem_w.at[t])
            pltpu.async_copy(ctile.at[t, f8], out_ce.at[h, f8, wid],
                             wsem_c.at[t])
        return carry

    lax.fori_loop(0, HIST, step, 0)

    for t in range(TSLOT):
        for f8 in range(8):
            pltpu.make_async_copy(wtile.at[t, f8], out_we.at[0, f8, 0],
                                  wsem_w.at[t]).wait()
            pltpu.make_async_copy(ctile.at[t, f8], out_ce.at[0, f8, 0],
                                  wsem_c.at[t]).wait()

    pltpu.make_async_copy(wbias_h.at[pl.ds(0, HIST * CH)],
                          wbias_v, bsem).wait()
    pltpu.make_async_copy(cbias_h.at[pl.ds(0, HIST * CH)],
                          cbias_v, bsem).wait()
    pltpu.sync_copy(wbias_v, out_wb.at[:, pl.ds(b0, CH)])
    pltpu.sync_copy(cbias_v, out_cb.at[:, pl.ds(b0, CH)])


@jax.jit
def _glove_sc(wordsT, ctxT, w_embeddings, w_biases, c_embeddings, c_biases):
    mesh = plsc.VectorSubcoreMesh(core_axis_name="c", subcore_axis_name="s",
                                  num_cores=NC, num_subcores=NS)
    f32 = jnp.float32
    run = pl.kernel(
        _glove_body,
        out_type=(
            jax.ShapeDtypeStruct((HIST, 8, NW, 8, CH), f32),
            jax.ShapeDtypeStruct((HIST, BATCH), f32),
            jax.ShapeDtypeStruct((HIST, 8, NW, 8, CH), f32),
            jax.ShapeDtypeStruct((HIST, BATCH), f32),
        ),
        mesh=mesh,
        compiler_params=pltpu.CompilerParams(use_tc_tiling_on_sc=False,
                                             needs_layout_passes=False),
        scratch_types=[
            pltpu.VMEM((HIST, CH), jnp.int32),
            pltpu.VMEM((HIST, CH), jnp.int32),
            pltpu.VMEM((NSLOT, CH, EMBED_DIM), f32),
            pltpu.VMEM((NSLOT, CH, EMBED_DIM), f32),
            pltpu.VMEM((TSLOT, 8, 8, CH), f32),
            pltpu.VMEM((TSLOT, 8, 8, CH), f32),
            pltpu.VMEM((HIST, CH), f32),
            pltpu.VMEM((HIST, CH), f32),
            pltpu.SemaphoreType.DMA((NSLOT,)),
            pltpu.SemaphoreType.DMA((NSLOT,)),
            pltpu.SemaphoreType.DMA((TSLOT,)),
            pltpu.SemaphoreType.DMA((TSLOT,)),
            pltpu.SemaphoreType.DMA,
        ],
    )
    return run(wordsT, ctxT, w_embeddings, w_biases, c_embeddings, c_biases)


def kernel(words, contexts, w_embeddings, w_biases, c_embeddings, c_biases):
    wordsT = words.astype(jnp.int32).T
    ctxT = contexts.astype(jnp.int32).T
    we5, wb, ce5, cb = _glove_sc(wordsT, ctxT,
                                 w_embeddings, w_biases.reshape(VOCAB),
                                 c_embeddings, c_biases.reshape(VOCAB))

    def chain(x5):
        return jnp.transpose(x5, (2, 4, 0, 1, 3)).reshape(BATCH, HIST,
                                                          EMBED_DIM)

    return (
        chain(we5),
        wb.T.reshape(BATCH, HIST, 1),
        chain(ce5),
        cb.T.reshape(BATCH, HIST, 1),
    )


# speedup vs baseline: 1.2026x; 1.2026x over previous
import jax
import jax.numpy as jnp
from jax import lax
from jax.experimental import pallas as pl
from jax.experimental.pallas import tpu as pltpu
from jax.experimental.pallas import tpu_sc as plsc

VOCAB = 100000
EMBED_DIM = 64
BATCH = 4096
HIST = 50

NC = 2
NS = 16
NW = NC * NS

CH = BATCH // NW
CHUNK_W = CH * EMBED_DIM
NSLOT = 3
TSLOT = 2
LOOKAHEAD = 2
LANES = 16


def _transpose_chunk(gbuf_s, tflat, scatbases):
    def per_b(b, carry):
        for k in range(EMBED_DIM // LANES):
            v = gbuf_s[b, pl.ds(k * LANES, LANES)]
            plsc.store_scatter(tflat, [scatbases[k] + b], v)
        return carry
    lax.fori_loop(0, CH, per_b, 0, unroll=4)


def _glove_body(words_h, ctx_h, wemb_h, wbias_h, cemb_h, cbias_h,
                out_we, out_wb, out_ce, out_cb,
                idx_w, idx_c, wbuf, cbuf, wtile, ctile, wbias_v, cbias_v,
                gsem_w, gsem_c, wsem_w, wsem_c, bsem):
    wid = lax.axis_index("s") * NC + lax.axis_index("c")
    b0 = wid * CH

    pltpu.sync_copy(words_h.at[:, pl.ds(b0, CH)], idx_w)
    pltpu.sync_copy(ctx_h.at[:, pl.ds(b0, CH)], idx_c)

    scatbases = [(lax.iota(jnp.int32, LANES) + 16 * k) * CH
                 for k in range(EMBED_DIM // LANES)]

    for h in range(LOOKAHEAD):
        pltpu.async_copy(wemb_h.at[idx_w.at[h]], wbuf.at[h], gsem_w.at[h])
        pltpu.async_copy(cemb_h.at[idx_c.at[h]], cbuf.at[h], gsem_c.at[h])

    def do_chunk(h, t):
        s = lax.rem(h, NSLOT)
        pltpu.make_async_copy(wemb_h.at[idx_w.at[h]], wbuf.at[s],
                              gsem_w.at[s]).wait()
        pltpu.make_async_copy(cemb_h.at[idx_c.at[h]], cbuf.at[s],
                              gsem_c.at[s]).wait()
        hn = h + LOOKAHEAD
        sn = lax.rem(hn, NSLOT)

        @pl.when(hn < HIST)
        def _():
            pltpu.async_copy(wemb_h.at[idx_w.at[hn]], wbuf.at[sn],
                             gsem_w.at[sn])
            pltpu.async_copy(cemb_h.at[idx_c.at[hn]], cbuf.at[sn],
                             gsem_c.at[sn])
        pltpu.async_copy(wbias_h.at[idx_w.at[h]], wbias_v.at[h], bsem)
        pltpu.async_copy(cbias_h.at[idx_c.at[h]], cbias_v.at[h], bsem)

        @pl.when(h >= TSLOT)
        def _():
            for f8 in range(8):
                pltpu.make_async_copy(wtile.at[t, pl.ds(f8 * 1024, 1024)],
                                      out_we.at[0, f8, 0],
                                      wsem_w.at[t]).wait()
                pltpu.make_async_copy(ctile.at[t, pl.ds(f8 * 1024, 1024)],
                                      out_ce.at[0, f8, 0],
                                      wsem_c.at[t]).wait()

        _transpose_chunk(wbuf.at[s], wtile.at[t], scatbases)
        _transpose_chunk(cbuf.at[s], ctile.at[t], scatbases)
        for f8 in range(8):
            pltpu.async_copy(wtile.at[t, pl.ds(f8 * 1024, 1024)],
                             out_we.at[h, f8, wid], wsem_w.at[t])
            pltpu.async_copy(ctile.at[t, pl.ds(f8 * 1024, 1024)],
                             out_ce.at[h, f8, wid], wsem_c.at[t])

    def step(hh, carry):
        do_chunk(hh * 2, 0)
        do_chunk(hh * 2 + 1, 1)
        return carry

    lax.fori_loop(0, HIST // 2, step, 0)

    for t in range(TSLOT):
        for f8 in range(8):
            pltpu.make_async_copy(wtile.at[t, pl.ds(f8 * 1024, 1024)],
                                  out_we.at[0, f8, 0], wsem_w.at[t]).wait()
            pltpu.make_async_copy(ctile.at[t, pl.ds(f8 * 1024, 1024)],
                                  out_ce.at[0, f8, 0], wsem_c.at[t]).wait()

    pltpu.make_async_copy(wbias_h.at[pl.ds(0, HIST * CH)],
                          wbias_v, bsem).wait()
    pltpu.make_async_copy(cbias_h.at[pl.ds(0, HIST * CH)],
                          cbias_v, bsem).wait()
    pltpu.sync_copy(wbias_v, out_wb.at[:, pl.ds(b0, CH)])
    pltpu.sync_copy(cbias_v, out_cb.at[:, pl.ds(b0, CH)])


@jax.jit
def _glove_sc(wordsT, ctxT, w_embeddings, w_biases, c_embeddings, c_biases):
    mesh = plsc.VectorSubcoreMesh(core_axis_name="c", subcore_axis_name="s",
                                  num_cores=NC, num_subcores=NS)
    f32 = jnp.float32
    run = pl.kernel(
        _glove_body,
        out_type=(
            jax.ShapeDtypeStruct((HIST, 8, NW, 1024), f32),
            jax.ShapeDtypeStruct((HIST, BATCH), f32),
            jax.ShapeDtypeStruct((HIST, 8, NW, 1024), f32),
            jax.ShapeDtypeStruct((HIST, BATCH), f32),
        ),
        mesh=mesh,
        compiler_params=pltpu.CompilerParams(use_tc_tiling_on_sc=False,
                                             needs_layout_passes=False),
        scratch_types=[
            pltpu.VMEM((HIST, CH), jnp.int32),
            pltpu.VMEM((HIST, CH), jnp.int32),
            pltpu.VMEM((NSLOT, CH, EMBED_DIM), f32),
            pltpu.VMEM((NSLOT, CH, EMBED_DIM), f32),
            pltpu.VMEM((TSLOT, CHUNK_W), f32),
            pltpu.VMEM((TSLOT, CHUNK_W), f32),
            pltpu.VMEM((HIST, CH), f32),
            pltpu.VMEM((HIST, CH), f32),
            pltpu.SemaphoreType.DMA((NSLOT,)),
            pltpu.SemaphoreType.DMA((NSLOT,)),
            pltpu.SemaphoreType.DMA((TSLOT,)),
            pltpu.SemaphoreType.DMA((TSLOT,)),
            pltpu.SemaphoreType.DMA,
        ],
    )
    return run(wordsT, ctxT, w_embeddings, w_biases, c_embeddings, c_biases)


def kernel(words, contexts, w_embeddings, w_biases, c_embeddings, c_biases):
    wordsT = words.astype(jnp.int32).T
    ctxT = contexts.astype(jnp.int32).T
    we5, wb, ce5, cb = _glove_sc(wordsT, ctxT,
                                 w_embeddings, w_biases.reshape(VOCAB),
                                 c_embeddings, c_biases.reshape(VOCAB))

    def chain(x4):
        x5 = x4.reshape(HIST, 8, NW, 8, CH)
        return jnp.transpose(x5, (2, 4, 0, 1, 3)).reshape(BATCH, HIST,
                                                          EMBED_DIM)

    return (
        chain(we5),
        wb.T.reshape(BATCH, HIST, 1),
        chain(ce5),
        cb.T.reshape(BATCH, HIST, 1),
    )


# speedup vs baseline: 1.2081x; 1.0046x over previous
import jax
import jax.numpy as jnp
from jax import lax
from jax.experimental import pallas as pl
from jax.experimental.pallas import tpu as pltpu
from jax.experimental.pallas import tpu_sc as plsc

VOCAB = 100000
EMBED_DIM = 64
BATCH = 4096
HIST = 50

NC = 2
NS = 16
NW = NC * NS

CH = BATCH // NW
CHUNK_W = CH * EMBED_DIM
NSLOT = 3
TSLOT = 2
LOOKAHEAD = 2
LANES = 16


def _transpose_chunks(wbuf_s, cbuf_s, wflat, cflat, scatbases):
    def per_b(b, carry):
        idxs = [scatbases[k] + b for k in range(EMBED_DIM // LANES)]
        vws = [wbuf_s[b, pl.ds(k * LANES, LANES)]
               for k in range(EMBED_DIM // LANES)]
        vcs = [cbuf_s[b, pl.ds(k * LANES, LANES)]
               for k in range(EMBED_DIM // LANES)]
        for k in range(EMBED_DIM // LANES):
            plsc.store_scatter(wflat, [idxs[k]], vws[k])
            plsc.store_scatter(cflat, [idxs[k]], vcs[k])
        return carry
    lax.fori_loop(0, CH, per_b, 0, unroll=4)


def _glove_body(words_h, ctx_h, wemb_h, wbias_h, cemb_h, cbias_h,
                out_we, out_wb, out_ce, out_cb,
                idx_w, idx_c, wbuf, cbuf, wtile, ctile, wbias_v, cbias_v,
                gsem_w, gsem_c, wsem_w, wsem_c, bsem):
    wid = lax.axis_index("s") * NC + lax.axis_index("c")
    b0 = wid * CH

    pltpu.sync_copy(words_h.at[:, pl.ds(b0, CH)], idx_w)
    pltpu.sync_copy(ctx_h.at[:, pl.ds(b0, CH)], idx_c)

    scatbases = [(lax.iota(jnp.int32, LANES) + 16 * k) * CH
                 for k in range(EMBED_DIM // LANES)]

    for h in range(LOOKAHEAD):
        pltpu.async_copy(wemb_h.at[idx_w.at[h]], wbuf.at[h], gsem_w.at[h])
        pltpu.async_copy(cemb_h.at[idx_c.at[h]], cbuf.at[h], gsem_c.at[h])

    def do_chunk(h, t):
        s = lax.rem(h, NSLOT)
        pltpu.make_async_copy(wemb_h.at[idx_w.at[h]], wbuf.at[s],
                              gsem_w.at[s]).wait()
        pltpu.make_async_copy(cemb_h.at[idx_c.at[h]], cbuf.at[s],
                              gsem_c.at[s]).wait()
        hn = h + LOOKAHEAD
        sn = lax.rem(hn, NSLOT)

        @pl.when(hn < HIST)
        def _():
            pltpu.async_copy(wemb_h.at[idx_w.at[hn]], wbuf.at[sn],
                             gsem_w.at[sn])
            pltpu.async_copy(cemb_h.at[idx_c.at[hn]], cbuf.at[sn],
                             gsem_c.at[sn])
        pltpu.async_copy(wbias_h.at[idx_w.at[h]], wbias_v.at[h], bsem)
        pltpu.async_copy(cbias_h.at[idx_c.at[h]], cbias_v.at[h], bsem)

        @pl.when(h >= TSLOT)
        def _():
            for f8 in range(8):
                pltpu.make_async_copy(wtile.at[t, pl.ds(f8 * 1024, 1024)],
                                      out_we.at[0, f8, 0],
                                      wsem_w.at[t]).wait()
                pltpu.make_async_copy(ctile.at[t, pl.ds(f8 * 1024, 1024)],
                                      out_ce.at[0, f8, 0],
                                      wsem_c.at[t]).wait()

        _transpose_chunks(wbuf.at[s], cbuf.at[s], wtile.at[t], ctile.at[t],
                          scatbases)
        for f8 in range(8):
            pltpu.async_copy(wtile.at[t, pl.ds(f8 * 1024, 1024)],
                             out_we.at[h, f8, wid], wsem_w.at[t])
            pltpu.async_copy(ctile.at[t, pl.ds(f8 * 1024, 1024)],
                             out_ce.at[h, f8, wid], wsem_c.at[t])

    def step(hh, carry):
        do_chunk(hh * 2, 0)
        do_chunk(hh * 2 + 1, 1)
        return carry

    lax.fori_loop(0, HIST // 2, step, 0)

    for t in range(TSLOT):
        for f8 in range(8):
            pltpu.make_async_copy(wtile.at[t, pl.ds(f8 * 1024, 1024)],
                                  out_we.at[0, f8, 0], wsem_w.at[t]).wait()
            pltpu.make_async_copy(ctile.at[t, pl.ds(f8 * 1024, 1024)],
                                  out_ce.at[0, f8, 0], wsem_c.at[t]).wait()

    pltpu.make_async_copy(wbias_h.at[pl.ds(0, HIST * CH)],
                          wbias_v, bsem).wait()
    pltpu.make_async_copy(cbias_h.at[pl.ds(0, HIST * CH)],
                          cbias_v, bsem).wait()
    pltpu.sync_copy(wbias_v, out_wb.at[:, pl.ds(b0, CH)])
    pltpu.sync_copy(cbias_v, out_cb.at[:, pl.ds(b0, CH)])


@jax.jit
def _glove_sc(wordsT, ctxT, w_embeddings, w_biases, c_embeddings, c_biases):
    mesh = plsc.VectorSubcoreMesh(core_axis_name="c", subcore_axis_name="s",
                                  num_cores=NC, num_subcores=NS)
    f32 = jnp.float32
    run = pl.kernel(
        _glove_body,
        out_type=(
            jax.ShapeDtypeStruct((HIST, 8, NW, 1024), f32),
            jax.ShapeDtypeStruct((HIST, BATCH), f32),
            jax.ShapeDtypeStruct((HIST, 8, NW, 1024), f32),
            jax.ShapeDtypeStruct((HIST, BATCH), f32),
        ),
        mesh=mesh,
        compiler_params=pltpu.CompilerParams(use_tc_tiling_on_sc=False,
                                             needs_layout_passes=False),
        scratch_types=[
            pltpu.VMEM((HIST, CH), jnp.int32),
            pltpu.VMEM((HIST, CH), jnp.int32),
            pltpu.VMEM((NSLOT, CH, EMBED_DIM), f32),
            pltpu.VMEM((NSLOT, CH, EMBED_DIM), f32),
            pltpu.VMEM((TSLOT, CHUNK_W), f32),
            pltpu.VMEM((TSLOT, CHUNK_W), f32),
            pltpu.VMEM((HIST, CH), f32),
            pltpu.VMEM((HIST, CH), f32),
            pltpu.SemaphoreType.DMA((NSLOT,)),
            pltpu.SemaphoreType.DMA((NSLOT,)),
            pltpu.SemaphoreType.DMA((TSLOT,)),
            pltpu.SemaphoreType.DMA((TSLOT,)),
            pltpu.SemaphoreType.DMA,
        ],
    )
    return run(wordsT, ctxT, w_embeddings, w_biases, c_embeddings, c_biases)


def kernel(words, contexts, w_embeddings, w_biases, c_embeddings, c_biases):
    wordsT = words.astype(jnp.int32).T
    ctxT = contexts.astype(jnp.int32).T
    we5, wb, ce5, cb = _glove_sc(wordsT, ctxT,
                                 w_embeddings, w_biases.reshape(VOCAB),
                                 c_embeddings, c_biases.reshape(VOCAB))

    def chain(x4):
        x5 = x4.reshape(HIST, 8, NW, 8, CH)
        return jnp.transpose(x5, (2, 4, 0, 1, 3)).reshape(BATCH, HIST,
                                                          EMBED_DIM)

    return (
        chain(we5),
        wb.T.reshape(BATCH, HIST, 1),
        chain(ce5),
        cb.T.reshape(BATCH, HIST, 1),
    )


# speedup vs baseline: 1.8899x; 1.5644x over previous
import functools

import jax
import jax.numpy as jnp
from jax import lax
from jax.experimental import pallas as pl
from jax.experimental.pallas import tpu as pltpu
from jax.experimental.pallas import tpu_sc as plsc

VOCAB = 100000
EMBED_DIM = 64
BATCH = 4096
HIST = 50

NC = 2
NS = 16
NW = NC * NS

CH = BATCH // NW
NSLOT = 6
LOOKAHEAD = 3


def _lookup_body(idxT_h, emb_h, bias_h, out_e, out_b,
                 idx_v, ebuf, bias_v, gsem, wsem, bsem):
    wid = lax.axis_index("s") * NC + lax.axis_index("c")
    b0 = wid * CH

    pltpu.sync_copy(idxT_h.at[:, pl.ds(b0, CH)], idx_v)

    for h in range(LOOKAHEAD):
        pltpu.async_copy(emb_h.at[idx_v.at[h]], ebuf.at[h], gsem.at[h])

    def step(h, carry):
        s = lax.rem(h, NSLOT)
        pltpu.make_async_copy(emb_h.at[idx_v.at[h]], ebuf.at[s],
                              gsem.at[s]).wait()
        pltpu.async_copy(ebuf.at[s], out_e.at[h, pl.ds(b0, CH)], wsem.at[s])
        pltpu.async_copy(bias_h.at[idx_v.at[h]], bias_v.at[h], bsem)
        hn = h + LOOKAHEAD
        sn = lax.rem(hn, NSLOT)

        @pl.when(hn < HIST)
        def _():
            @pl.when(hn >= NSLOT)
            def _():
                pltpu.make_async_copy(ebuf.at[sn],
                                      out_e.at[hn - NSLOT, pl.ds(b0, CH)],
                                      wsem.at[sn]).wait()
            pltpu.async_copy(emb_h.at[idx_v.at[hn]], ebuf.at[sn],
                             gsem.at[sn])
        return carry

    lax.fori_loop(0, HIST, step, 0)

    for s in range(NSLOT):
        pltpu.make_async_copy(ebuf.at[s],
                              out_e.at[HIST - NSLOT + s, pl.ds(b0, CH)],
                              wsem.at[s]).wait()

    pltpu.make_async_copy(bias_h.at[pl.ds(0, HIST * CH)], bias_v,
                          bsem).wait()
    pltpu.sync_copy(bias_v, out_b.at[:, pl.ds(b0, CH)])


def _lookup_call(idxT, emb, bias):
    mesh = plsc.VectorSubcoreMesh(core_axis_name="c", subcore_axis_name="s",
                                  num_cores=NC, num_subcores=NS)
    f32 = jnp.float32
    run = pl.kernel(
        _lookup_body,
        out_type=(
            jax.ShapeDtypeStruct((HIST, BATCH, EMBED_DIM), f32),
            jax.ShapeDtypeStruct((HIST, BATCH), f32),
        ),
        mesh=mesh,
        compiler_params=pltpu.CompilerParams(use_tc_tiling_on_sc=False),
        scratch_types=[
            pltpu.VMEM((HIST, CH), jnp.int32),
            pltpu.VMEM((NSLOT, CH, EMBED_DIM), f32),
            pltpu.VMEM((HIST, CH), f32),
            pltpu.SemaphoreType.DMA((NSLOT,)),
            pltpu.SemaphoreType.DMA((NSLOT,)),
            pltpu.SemaphoreType.DMA,
        ],
    )
    return run(idxT, emb, bias)


@jax.jit
def _glove_sc(wordsT, ctxT, w_embeddings, w_biases, c_embeddings, c_biases):
    we, wb = _lookup_call(wordsT, w_embeddings, w_biases)
    ce, cb = _lookup_call(ctxT, c_embeddings, c_biases)
    return we, wb, ce, cb


def kernel(words, contexts, w_embeddings, w_biases, c_embeddings, c_biases):
    wordsT = words.astype(jnp.int32).T
    ctxT = contexts.astype(jnp.int32).T
    we, wb, ce, cb = _glove_sc(wordsT, ctxT,
                               w_embeddings, w_biases.reshape(VOCAB),
                               c_embeddings, c_biases.reshape(VOCAB))
    return (
        jnp.transpose(we, (1, 0, 2)),
        wb.T.reshape(BATCH, HIST, 1),
        jnp.transpose(ce, (1, 0, 2)),
        cb.T.reshape(BATCH, HIST, 1),
    )


# speedup vs baseline: 2.5654x; 1.3574x over previous
import jax
import jax.numpy as jnp
from jax import lax
from jax.experimental import pallas as pl
from jax.experimental.pallas import tpu as pltpu
from jax.experimental.pallas import tpu_sc as plsc

VOCAB = 100000
EMBED_DIM = 64
BATCH = 4096
HIST = 50

NC = 2
NS = 16
NW = NC * NS

CH = BATCH // NW
NSLOT = 4
TSLOT = 2
LOOKAHEAD = 3
LANES = 16
SKEW = CH + 1


def _transpose_chunk(ebuf_s, tile_t, rows16):
    def per_b(b, carry):
        bb = jnp.full((LANES,), b, jnp.int32)
        for k in range(EMBED_DIM // LANES):
            v = ebuf_s[b, pl.ds(k * LANES, LANES)]
            plsc.store_scatter(tile_t, [rows16[k], bb], v)
        return carry
    lax.fori_loop(0, CH, per_b, 0, unroll=4)


def _lookup_body(idxT_h, emb_h, bias_h, out_e, out_b,
                 idx_v, ebuf, tile, bias_v, gsem, wsem, bsem):
    wid = lax.axis_index("s") * NC + lax.axis_index("c")
    b0 = wid * CH

    pltpu.sync_copy(idxT_h.at[:, pl.ds(b0, CH)], idx_v)

    rows16 = [lax.iota(jnp.int32, LANES) + LANES * k
              for k in range(EMBED_DIM // LANES)]

    for h in range(LOOKAHEAD):
        pltpu.async_copy(emb_h.at[idx_v.at[h]], ebuf.at[h], gsem.at[h])

    def do_chunk(h, t):
        s = lax.rem(h, NSLOT)
        pltpu.make_async_copy(emb_h.at[idx_v.at[h]], ebuf.at[s],
                              gsem.at[s]).wait()
        hn = h + LOOKAHEAD
        sn = lax.rem(hn, NSLOT)

        @pl.when(hn < HIST)
        def _():
            pltpu.async_copy(emb_h.at[idx_v.at[hn]], ebuf.at[sn],
                             gsem.at[sn])
        pltpu.async_copy(bias_h.at[idx_v.at[h]], bias_v.at[h], bsem)

        @pl.when(h >= TSLOT)
        def _():
            for f8 in range(8):
                pltpu.make_async_copy(
                    tile.at[t, pl.ds(f8 * 8, 8), pl.ds(0, CH)],
                    out_e.at[0, f8, 0], wsem.at[t]).wait()

        _transpose_chunk(ebuf.at[s], tile.at[t], rows16)
        for f8 in range(8):
            pltpu.async_copy(tile.at[t, pl.ds(f8 * 8, 8), pl.ds(0, CH)],
                             out_e.at[h, f8, wid], wsem.at[t])

    def step(hh, carry):
        do_chunk(hh * 2, 0)
        do_chunk(hh * 2 + 1, 1)
        return carry

    lax.fori_loop(0, HIST // 2, step, 0)

    for t in range(TSLOT):
        for f8 in range(8):
            pltpu.make_async_copy(tile.at[t, pl.ds(f8 * 8, 8), pl.ds(0, CH)],
                                  out_e.at[0, f8, 0], wsem.at[t]).wait()

    pltpu.make_async_copy(bias_h.at[pl.ds(0, HIST * CH)], bias_v,
                          bsem).wait()
    pltpu.sync_copy(bias_v, out_b.at[:, pl.ds(b0, CH)])


def _lookup_call(idxT, emb, bias):
    mesh = plsc.VectorSubcoreMesh(core_axis_name="c", subcore_axis_name="s",
                                  num_cores=NC, num_subcores=NS)
    f32 = jnp.float32
    run = pl.kernel(
        _lookup_body,
        out_type=(
            jax.ShapeDtypeStruct((HIST, 8, NW, 8, CH), f32),
            jax.ShapeDtypeStruct((HIST, BATCH), f32),
        ),
        mesh=mesh,
        compiler_params=pltpu.CompilerParams(use_tc_tiling_on_sc=False,
                                             needs_layout_passes=False),
        scratch_types=[
            pltpu.VMEM((HIST, CH), jnp.int32),
            pltpu.VMEM((NSLOT, CH, EMBED_DIM), f32),
            pltpu.VMEM((TSLOT, EMBED_DIM, SKEW), f32),
            pltpu.VMEM((HIST, CH), f32),
            pltpu.SemaphoreType.DMA((NSLOT,)),
            pltpu.SemaphoreType.DMA((TSLOT,)),
            pltpu.SemaphoreType.DMA,
        ],
    )
    return run(idxT, emb, bias)


@jax.jit
def _glove_sc(wordsT, ctxT, w_embeddings, w_biases, c_embeddings, c_biases):
    we, wb = _lookup_call(wordsT, w_embeddings, w_biases)
    ce, cb = _lookup_call(ctxT, c_embeddings, c_biases)
    return we, wb, ce, cb


def kernel(words, contexts, w_embeddings, w_biases, c_embeddings, c_biases):
    wordsT = words.astype(jnp.int32).T
    ctxT = contexts.astype(jnp.int32).T
    we5, wb, ce5, cb = _glove_sc(wordsT, ctxT,
                                 w_embeddings, w_biases.reshape(VOCAB),
                                 c_embeddings, c_biases.reshape(VOCAB))

    def chain(x5):
        return jnp.transpose(x5, (2, 4, 0, 1, 3)).reshape(BATCH, HIST,
                                                          EMBED_DIM)

    return (
        chain(we5),
        wb.T.reshape(BATCH, HIST, 1),
        chain(ce5),
        cb.T.reshape(BATCH, HIST, 1),
    )
